# baseline (speedup 1.0000x reference)
import functools

import jax
import jax.numpy as jnp
from jax import lax
from jax.experimental import pallas as pl
from jax.experimental.pallas import tpu as pltpu
from jax.experimental.pallas import tpu_sc as plsc

_N = 10000
_E = 320000
_IN = 128
_HID = 16
_HEADS = 8
_OUT = 64

_NC = 2
_NS = 16
_NW = _NC * _NS
_EPW = _E // _NW
_NPAD = 10000
_ROWS_PT = _NPAD // _NS
_ZSTEP = 125

_MROWS = 2000

_HI = lax.Precision.HIGHEST
_DN = (((1,), (0,)), ((), ()))


def _tc1_body(x_ref, w_ref, ms_ref, md_ref, h_ref, as_ref, ad_ref):
    h = lax.dot_general(x_ref[...], w_ref[...], _DN, precision=_HI)
    h_ref[...] = h
    as_ref[...] = lax.dot_general(h, ms_ref[...], _DN, precision=_HI)
    ad_ref[...] = lax.dot_general(h, md_ref[...], _DN, precision=_HI)


def _tc2_body(acc_ref, den_ref, e1_ref, b1_ref, w2_ref, m2s_ref, m2d_ref,
              h2_ref, as2_ref, ad2_ref):
    accsum = acc_ref[0] + acc_ref[1]
    densum = den_ref[0] + den_ref[1]
    den_exp = lax.dot_general(densum, e1_ref[...], _DN, precision=_HI)
    h = accsum / (den_exp + 1e-16) + b1_ref[...]
    h = jnp.where(h >= 0.0, h, 0.2 * h)
    h2 = lax.dot_general(h, w2_ref[...], _DN, precision=_HI)
    h2_ref[...] = h2.astype(h2_ref.dtype)
    as2_ref[...] = lax.dot_general(h2, m2s_ref[...], _DN, precision=_HI)
    ad2_ref[...] = lax.dot_general(h2, m2d_ref[...], _DN, precision=_HI)


def _tc3_body(acc_ref, den_ref, e2_ref, b2_ref, o_ref):
    accsum = acc_ref[0] + acc_ref[1]
    densum = den_ref[0] + den_ref[1]
    den_exp = lax.dot_general(densum, e2_ref[...], _DN, precision=_HI)
    o_ref[...] = accsum / (den_exp + 1e-16) + b2_ref[...]


def _make_edge_kernel(C, H, K, bf16_feat=False):
    nv = C // 16
    grp = (C // H) // 16
    KA = ((K // 2 + 7) // 8) * 8
    KB = K - KA
    nch = _EPW // K
    assert not bf16_feat or H == 1
    mesh = plsc.VectorSubcoreMesh(core_axis_name="c", subcore_axis_name="s",
                                  num_cores=_NC, num_subcores=_NS)

    @functools.partial(
        pl.kernel,
        out_type=[jax.ShapeDtypeStruct((_NC, _NPAD, C), jnp.float32),
                  jax.ShapeDtypeStruct((_NC, _NPAD, 16), jnp.float32)],
        mesh=mesh,
        compiler_params=pltpu.CompilerParams(
            use_tc_tiling_on_sc=False,
            needs_layout_passes=not bf16_feat),
        scratch_types=[
            pltpu.VMEM((K,), jnp.int32),
            pltpu.VMEM((K,), jnp.int32),
            pltpu.VMEM((K,), jnp.int32),
            pltpu.VMEM((K,), jnp.int32),
            pltpu.VMEM((KA,), jnp.int32),
            pltpu.VMEM((KB,), jnp.int32),
            pltpu.VMEM((KA,), jnp.int32),
            pltpu.VMEM((KB,), jnp.int32),
            pltpu.VMEM((K, 16), jnp.float32),
            pltpu.VMEM((K, 16), jnp.float32),
            pltpu.VMEM((K, 16), jnp.float32),
            pltpu.VMEM((K, 16), jnp.float32),
            pltpu.VMEM((K, C), jnp.float32),
            pltpu.VMEM((K, C // 2) if bf16_feat else (8, 16),
                       jnp.int32),
            pltpu.VMEM_SHARED((_NPAD, C), jnp.float32),
            pltpu.VMEM_SHARED((_NPAD, 16), jnp.float32),
            pltpu.SemaphoreType.DMA,
            pltpu.SemaphoreType.DMA,
            pltpu.SemaphoreType.DMA,
            pltpu.SemaphoreType.DMA,
            pltpu.SemaphoreType.DMA,
            pltpu.SemaphoreType.DMA,
            pltpu.SemaphoreType.DMA,
            pltpu.SemaphoreType.DMA,
            pltpu.SemaphoreType.DMA,
            pltpu.SemaphoreType.DMA,
        ],
    )
    def edge_kernel(h_hbm, asp_hbm, adp_hbm, src_hbm, dst_hbm,
                    acc_hbm, den_hbm,
                    idx_s0, idx_d0, idx_s1, idx_d1,
                    sia0, sib0, sia1, sib1,
                    gs0, gd0, gs1, gd1, hs, hsi, acc_sh, den_sh,
                    isem0, isem1, gsem0, gsem1,
                    hsemA, hsemB, dsem0, dsem1, asemA, asemB):
        c = lax.axis_index("c")
        s = lax.axis_index("s")
        wid = s * _NC + c
        ebase = wid * _EPW
        idxs = ((idx_s0, idx_d0), (idx_s1, idx_d1))
        sidxs = ((sia0, sib0), (sia1, sib1))
        gbufs = ((gs0, gd0), (gs1, gd1))
        isems = (isem0, isem1)
        gsems = (gsem0, gsem1)
        dsems = (dsem0, dsem1)
        hsA = hs.at[pl.ds(0, KA)]
        hsB = hs.at[pl.ds(KA, KB)]

        zv = jnp.zeros((16,), jnp.float32)

        @pl.loop(0, _ZSTEP)
        def _(k):
            gs0[k, :] = zv
            for j in range(nv):
                hs[k, pl.ds(j * 16, 16)] = zv

        base = s * _ROWS_PT

        @pl.loop(0, _ROWS_PT, step=_ZSTEP)
        def _(r):
            pltpu.sync_copy(hs.at[pl.ds(0, _ZSTEP)],
                            acc_sh.at[pl.ds(base + r, _ZSTEP)])
            pltpu.sync_copy(gs0.at[pl.ds(0, _ZSTEP)],
                            den_sh.at[pl.ds(base + r, _ZSTEP)])

        plsc.subcore_barrier()

        pltpu.sync_copy(src_hbm.at[pl.ds(ebase, K)], idx_s0)
        pltpu.sync_copy(dst_hbm.at[pl.ds(ebase, K)], idx_d0)
        pltpu.async_copy(asp_hbm.at[idx_s0], gs0, gsem0)
        pltpu.async_copy(adp_hbm.at[idx_d0], gd0, gsem0)
        pltpu.async_copy(src_hbm.at[pl.ds(ebase + K, K)], idx_s1, isem1)
        pltpu.async_copy(dst_hbm.at[pl.ds(ebase + K, K)], idx_d1, isem1)

        def drain_prev(b):
            sa2, sb2 = sidxs[1 - b]
            gs2, _gd2 = gbufs[1 - b]
            pltpu.make_async_copy(hsA, acc_sh.at[sa2], asemA).wait()
            pltpu.make_async_copy(hsB, acc_sh.at[sb2], asemB).wait()
            pltpu.make_async_copy(gs2.at[pl.ds(0, KA)],
                                  den_sh.at[sa2], dsems[1 - b]).wait()
            pltpu.make_async_copy(gs2.at[pl.ds(KA, KB)],
                                  den_sh.at[sb2], dsems[1 - b]).wait()

        def chunk_body(o, b):
            idx_s, idx_d = idxs[b]
            sia, sib = sidxs[b]
            gs, gd = gbufs[b]
            idx_s2, idx_d2 = idxs[1 - b]
            gs2, gd2 = gbufs[1 - b]

            @pl.when(o > 0)
            def _():
                drain_prev(b)

            if bf16_feat:
                cphA = pltpu.async_copy(h_hbm.at[idx_s.at[pl.ds(0, KA)]],
                                        hsi.at[pl.ds(0, KA)], hsemA)
                cphB = pltpu.async_copy(h_hbm.at[idx_s.at[pl.ds(KA, KB)]],
                                        hsi.at[pl.ds(KA, KB)], hsemB)
            else:
                cphA = pltpu.async_copy(h_hbm.at[idx_s.at[pl.ds(0, KA)]],
                                        hsA, hsemA)
                cphB = pltpu.async_copy(h_hbm.at[idx_s.at[pl.ds(KA, KB)]],
                                        hsB, hsemB)

            @pl.when(o + K < _EPW)
            def _():
                pltpu.make_async_copy(src_hbm.at[pl.ds(ebase, K)],
                                      idx_s2, isems[1 - b]).wait()
                pltpu.make_async_copy(dst_hbm.at[pl.ds(ebase, K)],
                                      idx_d2, isems[1 - b]).wait()
                pltpu.async_copy(asp_hbm.at[idx_s2], gs2, gsems[1 - b])
                pltpu.async_copy(adp_hbm.at[idx_d2], gd2, gsems[1 - b])

            for dst_ref, off, n in ((sia, 0, KA), (sib, KA, KB)):
                offs = list(range(0, n - 15, 16))
                if offs[-1] + 16 < n:
                    offs.append(n - 16)
                for oo in offs:
                    dst_ref[pl.ds(oo, 16)] = idx_d[pl.ds(off + oo, 16)]

            pltpu.make_async_copy(asp_hbm.at[idx_s], gs, gsems[b]).wait()
            pltpu.make_async_copy(adp_hbm.at[idx_d], gd, gsems[b]).wait()

            @pl.loop(0, K, step=4)
            def _(k):
                for u in range(4):
                    e = gs[k + u, :] + gd[k + u, :]
                    e = jnp.where(e >= 0.0, e, 0.2 * e)
                    gs[k + u, :] = jnp.exp(e)

            pltpu.async_copy(gs.at[pl.ds(0, KA)], den_sh.at[sia],
                             dsems[b], add=True)
            pltpu.async_copy(gs.at[pl.ds(KA, KB)], den_sh.at[sib],
                             dsems[b], add=True)

            def weight_rows(base_k, count):
                @pl.loop(0, count, step=2)
                def _(k):
                    for u in range(2):
                        kk = base_k + k + u
                        pv = gs[kk, :]
                        if bf16_feat:
                            pk = pv[0]
                            for hh in range(C // 32):
                                w = hsi[kk, pl.ds(hh * 16, 16)]
                                lo = plsc.bitcast(
                                    lax.shift_left(w, 16), jnp.float32)
                                hi = plsc.bitcast(
                                    jnp.bitwise_and(w, jnp.int32(-65536)),
                                    jnp.float32)
                                hs[kk, pl.ds(hh * 32, 16)] = lo * pk
                                hs[kk, pl.ds(hh * 32 + 16, 16)] = hi * pk
                        else:
                            for j in range(nv):
                                pk = pv[j // grp]
                                hs[kk, pl.ds(j * 16, 16)] = (
                                    hs[kk, pl.ds(j * 16, 16)] * pk)

            cphA.wait()
            weight_rows(0, KA)
            pltpu.async_copy(hsA, acc_sh.at[sia], asemA, add=True)
            cphB.wait()
            weight_rows(KA, KB)
            pltpu.async_copy(hsB, acc_sh.at[sib], asemB, add=True)

            @pl.when(o + 2 * K < _EPW)
            def _():
                pltpu.async_copy(src_hbm.at[pl.ds(ebase + o + 2 * K, K)],
                                 idx_s, isems[b])
                pltpu.async_copy(dst_hbm.at[pl.ds(ebase + o + 2 * K, K)],
                                 idx_d, isems[b])

        @pl.loop(0, _EPW - (K if nch % 2 else 0), step=2 * K)
        def _(o):
            chunk_body(o, 0)
            chunk_body(o + K, 1)

        if nch % 2:
            chunk_body(_EPW - K, 0)
            drain_prev(1)
        else:
            drain_prev(0)

        plsc.subcore_barrier()

        pltpu.sync_copy(acc_sh.at[pl.ds(base, _ROWS_PT)],
                        acc_hbm.at[c, pl.ds(base, _ROWS_PT)])
        pltpu.sync_copy(den_sh.at[pl.ds(base, _ROWS_PT)],
                        den_hbm.at[c, pl.ds(base, _ROWS_PT)])

    return edge_kernel


_edge_cache = {}


def _edge_kernel(C, H, K, bf16_feat=False):
    key = (C, H, K, bf16_feat)
    if key not in _edge_cache:
        _edge_cache[key] = _make_edge_kernel(C, H, K, bf16_feat)
    return _edge_cache[key]


def kernel(x, edge_index, W1, a_src1, a_dst1, b1, W2, a_src2, a_dst2, b2):
    src = edge_index[0]
    dst = edge_index[1]
    f32 = jnp.float32

    a1s = a_src1.reshape(_HEADS, _HID).astype(f32)
    a1d = a_dst1.reshape(_HEADS, _HID).astype(f32)
    eye8 = jnp.eye(_HEADS, dtype=f32)
    m1s = jnp.pad((a1s[:, :, None] * eye8[:, None, :]).reshape(_IN, _HEADS),
                  ((0, 0), (0, 16 - _HEADS)))
    m1d = jnp.pad((a1d[:, :, None] * eye8[:, None, :]).reshape(_IN, _HEADS),
                  ((0, 0), (0, 16 - _HEADS)))
    e1 = jnp.pad(jnp.kron(eye8, jnp.ones((1, _HID), f32)),
                 ((0, 16 - _HEADS), (0, 0)))
    e2 = jnp.pad(jnp.ones((1, _OUT), f32), ((0, 15), (0, 0)))
    perm = [32 * (j // 32) + (16 if j % 2 else 0) + (j % 32) // 2
            for j in range(_OUT)]
    W2p = W2[:, jnp.array(perm)]
    a2s = a_src2.reshape(_OUT)[jnp.array(perm)].reshape(_OUT, 1).astype(f32)
    a2d = a_dst2.reshape(_OUT)[jnp.array(perm)].reshape(_OUT, 1).astype(f32)
    m2s = jnp.pad(a2s, ((0, 0), (0, 15)))
    m2d = jnp.pad(a2d, ((0, 0), (0, 15)))

    grid = _N // _MROWS
    c1 = _HEADS * _HID

    tc_params = pltpu.CompilerParams(
        dimension_semantics=("parallel",))
    h1, as1, ad1 = pl.pallas_call(
        _tc1_body,
        grid=(grid,),
        compiler_params=tc_params,
        in_specs=[
            pl.BlockSpec((_MROWS, _IN), lambda i: (i, 0)),
            pl.BlockSpec((_IN, c1), lambda i: (0, 0)),
            pl.BlockSpec((c1, 16), lambda i: (0, 0)),
            pl.BlockSpec((c1, 16), lambda i: (0, 0)),
        ],
        out_specs=[
            pl.BlockSpec((_MROWS, c1), lambda i: (i, 0)),
            pl.BlockSpec((_MROWS, 16), lambda i: (i, 0)),
            pl.BlockSpec((_MROWS, 16), lambda i: (i, 0)),
        ],
        out_shape=[
            jax.ShapeDtypeStruct((_N, c1), f32),
            jax.ShapeDtypeStruct((_N, 16), f32),
            jax.ShapeDtypeStruct((_N, 16), f32),
        ],
    )(x, W1, m1s, m1d)

    acc1, den1 = _edge_kernel(c1, _HEADS, 200)(h1, as1, ad1, src, dst)

    h2, as2, ad2 = pl.pallas_call(
        _tc2_body,
        grid=(grid,),
        compiler_params=tc_params,
        in_specs=[
            pl.BlockSpec((_NC, _MROWS, c1), lambda i: (0, i, 0)),
            pl.BlockSpec((_NC, _MROWS, 16), lambda i: (0, i, 0)),
            pl.BlockSpec((16, c1), lambda i: (0, 0)),
            pl.BlockSpec((1, c1), lambda i: (0, 0)),
            pl.BlockSpec((c1, _OUT), lambda i: (0, 0)),
            pl.BlockSpec((_OUT, 16), lambda i: (0, 0)),
            pl.BlockSpec((_OUT, 16), lambda i: (0, 0)),
        ],
        out_specs=[
            pl.BlockSpec((_MROWS, _OUT), lambda i: (i, 0)),
            pl.BlockSpec((_MROWS, 16), lambda i: (i, 0)),
            pl.BlockSpec((_MROWS, 16), lambda i: (i, 0)),
        ],
        out_shape=[
            jax.ShapeDtypeStruct((_N, _OUT), jnp.bfloat16),
            jax.ShapeDtypeStruct((_N, 16), f32),
            jax.ShapeDtypeStruct((_N, 16), f32),
        ],
    )(acc1, den1, e1, b1.reshape(1, c1), W2p, m2s, m2d)
    h2i = lax.bitcast_convert_type(
        h2.reshape(_N, _OUT // 2, 2), jnp.int32)

    acc2, den2 = _edge_kernel(_OUT, 1, 400, True)(h2i, as2, ad2, src, dst)

    out = pl.pallas_call(
        _tc3_body,
        grid=(grid,),
        compiler_params=tc_params,
        in_specs=[
            pl.BlockSpec((_NC, _MROWS, _OUT), lambda i: (0, i, 0)),
            pl.BlockSpec((_NC, _MROWS, 16), lambda i: (0, i, 0)),
            pl.BlockSpec((16, _OUT), lambda i: (0, 0)),
            pl.BlockSpec((1, _OUT), lambda i: (0, 0)),
        ],
        out_specs=pl.BlockSpec((_MROWS, _OUT), lambda i: (i, 0)),
        out_shape=jax.ShapeDtypeStruct((_N, _OUT), f32),
    )(acc2, den2, e2, b2.reshape(1, _OUT))

    return out


# speedup vs baseline: 1.1033x; 1.1033x over previous
import functools

import jax
import jax.numpy as jnp
from jax import lax
from jax.experimental import pallas as pl
from jax.experimental.pallas import tpu as pltpu
from jax.experimental.pallas import tpu_sc as plsc

_N = 10000
_E = 320000
_IN = 128
_HID = 16
_HEADS = 8
_OUT = 64

_NC = 2
_NS = 16
_NW = _NC * _NS
_EPW = _E // _NW
_NPAD = 10000
_ROWS_PT = _NPAD // _NS
_ZSTEP = 125

_MROWS = 2000

_HI = lax.Precision.HIGHEST
_DN = (((1,), (0,)), ((), ()))


def _tc1_body(x_ref, w_ref, ms_ref, md_ref, h_ref, as_ref, ad_ref):
    h = lax.dot_general(x_ref[...], w_ref[...], _DN, precision=_HI)
    h_ref[...] = h
    as_ref[...] = lax.dot_general(h, ms_ref[...], _DN, precision=_HI)
    ad_ref[...] = lax.dot_general(h, md_ref[...], _DN, precision=_HI)


def _tc2_body(acc_ref, den_ref, e1_ref, b1_ref, w2_ref, m2s_ref, m2d_ref,
              h2_ref, as2_ref, ad2_ref):
    accsum = acc_ref[0] + acc_ref[1]
    densum = den_ref[0] + den_ref[1]
    den_exp = lax.dot_general(densum, e1_ref[...], _DN, precision=_HI)
    h = accsum / (den_exp + 1e-16) + b1_ref[...]
    h = jnp.where(h >= 0.0, h, 0.2 * h)
    h2 = lax.dot_general(h, w2_ref[...], _DN, precision=_HI)
    h2_ref[...] = h2.astype(h2_ref.dtype)
    as2_ref[...] = lax.dot_general(h2, m2s_ref[...], _DN, precision=_HI)
    ad2_ref[...] = lax.dot_general(h2, m2d_ref[...], _DN, precision=_HI)


def _tc3_body(acc_ref, den_ref, e2_ref, b2_ref, o_ref):
    accsum = acc_ref[0] + acc_ref[1]
    densum = den_ref[0] + den_ref[1]
    den_exp = lax.dot_general(densum, e2_ref[...], _DN, precision=_HI)
    o_ref[...] = accsum / (den_exp + 1e-16) + b2_ref[...]


def _make_edge_kernel(C, H, K, bf16_feat=False):
    nv = C // 16
    grp = (C // H) // 16
    KA = ((K // 2 + 7) // 8) * 8
    KB = K - KA
    nch = _EPW // K
    assert not bf16_feat or H == 1
    mesh = plsc.VectorSubcoreMesh(core_axis_name="c", subcore_axis_name="s",
                                  num_cores=_NC, num_subcores=_NS)

    @functools.partial(
        pl.kernel,
        out_type=[jax.ShapeDtypeStruct((_NC, _NPAD, C), jnp.float32),
                  jax.ShapeDtypeStruct((_NC, _NPAD, 16), jnp.float32)],
        mesh=mesh,
        compiler_params=pltpu.CompilerParams(
            use_tc_tiling_on_sc=False,
            needs_layout_passes=not bf16_feat),
        scratch_types=[
            pltpu.VMEM((K,), jnp.int32),
            pltpu.VMEM((K,), jnp.int32),
            pltpu.VMEM((K,), jnp.int32),
            pltpu.VMEM((K,), jnp.int32),
            pltpu.VMEM((KA,), jnp.int32),
            pltpu.VMEM((KB,), jnp.int32),
            pltpu.VMEM((KA,), jnp.int32),
            pltpu.VMEM((KB,), jnp.int32),
            pltpu.VMEM((K, 16), jnp.float32),
            pltpu.VMEM((K, 16), jnp.float32),
            pltpu.VMEM((K, 16), jnp.float32),
            pltpu.VMEM((K, 16), jnp.float32),
            pltpu.VMEM((K, C), jnp.float32),
            pltpu.VMEM((K, C // 2) if bf16_feat else (8, 16),
                       jnp.int32),
            pltpu.VMEM_SHARED((_NPAD, C), jnp.float32),
            pltpu.VMEM_SHARED((_NPAD, 16), jnp.float32),
            pltpu.SemaphoreType.DMA,
            pltpu.SemaphoreType.DMA,
            pltpu.SemaphoreType.DMA,
            pltpu.SemaphoreType.DMA,
            pltpu.SemaphoreType.DMA,
            pltpu.SemaphoreType.DMA,
            pltpu.SemaphoreType.DMA,
            pltpu.SemaphoreType.DMA,
            pltpu.SemaphoreType.DMA,
            pltpu.SemaphoreType.DMA,
        ],
    )
    def edge_kernel(h_hbm, asp_hbm, adp_hbm, src_hbm, dst_hbm,
                    acc_hbm, den_hbm,
                    idx_s0, idx_d0, idx_s1, idx_d1,
                    sia0, sib0, sia1, sib1,
                    gs0, gd0, gs1, gd1, hs, hsi, acc_sh, den_sh,
                    isem0, isem1, gsem0, gsem1,
                    hsemA, hsemB, dsem0, dsem1, asemA, asemB):
        c = lax.axis_index("c")
        s = lax.axis_index("s")
        wid = s * _NC + c
        ebase = wid * _EPW
        idxs = ((idx_s0, idx_d0), (idx_s1, idx_d1))
        sidxs = ((sia0, sib0), (sia1, sib1))
        gbufs = ((gs0, gd0), (gs1, gd1))
        isems = (isem0, isem1)
        gsems = (gsem0, gsem1)
        dsems = (dsem0, dsem1)
        hsA = hs.at[pl.ds(0, KA)]
        hsB = hs.at[pl.ds(KA, KB)]

        zv = jnp.zeros((16,), jnp.float32)

        @pl.loop(0, _ZSTEP)
        def _(k):
            gs0[k, :] = zv
            for j in range(nv):
                hs[k, pl.ds(j * 16, 16)] = zv

        base = s * _ROWS_PT

        @pl.loop(0, _ROWS_PT, step=_ZSTEP)
        def _(r):
            pltpu.sync_copy(hs.at[pl.ds(0, _ZSTEP)],
                            acc_sh.at[pl.ds(base + r, _ZSTEP)])
            pltpu.sync_copy(gs0.at[pl.ds(0, _ZSTEP)],
                            den_sh.at[pl.ds(base + r, _ZSTEP)])

        plsc.subcore_barrier()

        pltpu.sync_copy(src_hbm.at[pl.ds(ebase, K)], idx_s0)
        pltpu.sync_copy(dst_hbm.at[pl.ds(ebase, K)], idx_d0)
        pltpu.async_copy(asp_hbm.at[idx_s0], gs0, gsem0)
        pltpu.async_copy(adp_hbm.at[idx_d0], gd0, gsem0)
        pltpu.async_copy(src_hbm.at[pl.ds(ebase + K, K)], idx_s1, isem1)
        pltpu.async_copy(dst_hbm.at[pl.ds(ebase + K, K)], idx_d1, isem1)

        def drain_prev(b):
            sa2, sb2 = sidxs[1 - b]
            gs2, _gd2 = gbufs[1 - b]
            pltpu.make_async_copy(hsA, acc_sh.at[sa2], asemA).wait()
            pltpu.make_async_copy(hsB, acc_sh.at[sb2], asemB).wait()
            pltpu.make_async_copy(gs2.at[pl.ds(0, KA)],
                                  den_sh.at[sa2], dsems[1 - b]).wait()
            pltpu.make_async_copy(gs2.at[pl.ds(KA, KB)],
                                  den_sh.at[sb2], dsems[1 - b]).wait()

        def chunk_body(o, b):
            idx_s, idx_d = idxs[b]
            sia, sib = sidxs[b]
            gs, gd = gbufs[b]
            idx_s2, idx_d2 = idxs[1 - b]
            gs2, gd2 = gbufs[1 - b]

            @pl.when(o > 0)
            def _():
                drain_prev(b)

            if bf16_feat:
                cphA = pltpu.async_copy(h_hbm.at[idx_s.at[pl.ds(0, KA)]],
                                        hsi.at[pl.ds(0, KA)], hsemA)
                cphB = pltpu.async_copy(h_hbm.at[idx_s.at[pl.ds(KA, KB)]],
                                        hsi.at[pl.ds(KA, KB)], hsemB)
            else:
                cphA = pltpu.async_copy(h_hbm.at[idx_s.at[pl.ds(0, KA)]],
                                        hsA, hsemA)
                cphB = pltpu.async_copy(h_hbm.at[idx_s.at[pl.ds(KA, KB)]],
                                        hsB, hsemB)

            @pl.when(o + K < _EPW)
            def _():
                pltpu.make_async_copy(src_hbm.at[pl.ds(ebase, K)],
                                      idx_s2, isems[1 - b]).wait()
                pltpu.make_async_copy(dst_hbm.at[pl.ds(ebase, K)],
                                      idx_d2, isems[1 - b]).wait()
                pltpu.async_copy(asp_hbm.at[idx_s2], gs2, gsems[1 - b])
                pltpu.async_copy(adp_hbm.at[idx_d2], gd2, gsems[1 - b])

            for dst_ref, off, n in ((sia, 0, KA), (sib, KA, KB)):
                offs = list(range(0, n - 15, 16))
                if offs[-1] + 16 < n:
                    offs.append(n - 16)
                for oo in offs:
                    dst_ref[pl.ds(oo, 16)] = idx_d[pl.ds(off + oo, 16)]

            pltpu.make_async_copy(asp_hbm.at[idx_s], gs, gsems[b]).wait()
            pltpu.make_async_copy(adp_hbm.at[idx_d], gd, gsems[b]).wait()

            @pl.loop(0, K, step=4)
            def _(k):
                for u in range(4):
                    e = gs[k + u, :] + gd[k + u, :]
                    e = jnp.where(e >= 0.0, e, 0.2 * e)
                    gs[k + u, :] = jnp.exp(e)

            pltpu.async_copy(gs.at[pl.ds(0, KA)], den_sh.at[sia],
                             dsems[b], add=True)
            pltpu.async_copy(gs.at[pl.ds(KA, KB)], den_sh.at[sib],
                             dsems[b], add=True)

            def weight_rows(base_k, count):
                @pl.loop(0, count, step=2)
                def _(k):
                    for u in range(2):
                        kk = base_k + k + u
                        pv = gs[kk, :]
                        if bf16_feat:
                            pk = pv[0]
                            for hh in range(C // 32):
                                w = hsi[kk, pl.ds(hh * 16, 16)]
                                lo = plsc.bitcast(
                                    lax.shift_left(w, 16), jnp.float32)
                                hi = plsc.bitcast(
                                    jnp.bitwise_and(w, jnp.int32(-65536)),
                                    jnp.float32)
                                hs[kk, pl.ds(hh * 32, 16)] = lo * pk
                                hs[kk, pl.ds(hh * 32 + 16, 16)] = hi * pk
                        else:
                            for j in range(nv):
                                pk = pv[j // grp]
                                hs[kk, pl.ds(j * 16, 16)] = (
                                    hs[kk, pl.ds(j * 16, 16)] * pk)

            cphA.wait()
            weight_rows(0, KA)
            pltpu.async_copy(hsA, acc_sh.at[sia], asemA, add=True)
            cphB.wait()
            weight_rows(KA, KB)
            pltpu.async_copy(hsB, acc_sh.at[sib], asemB, add=True)

            @pl.when(o + 2 * K < _EPW)
            def _():
                pltpu.async_copy(src_hbm.at[pl.ds(ebase + o + 2 * K, K)],
                                 idx_s, isems[b])
                pltpu.async_copy(dst_hbm.at[pl.ds(ebase + o + 2 * K, K)],
                                 idx_d, isems[b])

        @pl.loop(0, _EPW - (K if nch % 2 else 0), step=2 * K)
        def _(o):
            chunk_body(o, 0)
            chunk_body(o + K, 1)

        if nch % 2:
            chunk_body(_EPW - K, 0)
            drain_prev(1)
        else:
            drain_prev(0)

        plsc.subcore_barrier()

        pltpu.sync_copy(acc_sh.at[pl.ds(base, _ROWS_PT)],
                        acc_hbm.at[c, pl.ds(base, _ROWS_PT)])
        pltpu.sync_copy(den_sh.at[pl.ds(base, _ROWS_PT)],
                        den_hbm.at[c, pl.ds(base, _ROWS_PT)])

    return edge_kernel


_edge_cache = {}


def _edge_kernel(C, H, K, bf16_feat=False):
    key = (C, H, K, bf16_feat)
    if key not in _edge_cache:
        _edge_cache[key] = _make_edge_kernel(C, H, K, bf16_feat)
    return _edge_cache[key]


def kernel(x, edge_index, W1, a_src1, a_dst1, b1, W2, a_src2, a_dst2, b2):
    src = edge_index[0]
    dst = edge_index[1]
    f32 = jnp.float32

    a1s = a_src1.reshape(_HEADS, _HID).astype(f32)
    a1d = a_dst1.reshape(_HEADS, _HID).astype(f32)
    eye8 = jnp.eye(_HEADS, dtype=f32)
    m1s = jnp.pad((a1s[:, :, None] * eye8[:, None, :]).reshape(_IN, _HEADS),
                  ((0, 0), (0, 16 - _HEADS)))
    m1d = jnp.pad((a1d[:, :, None] * eye8[:, None, :]).reshape(_IN, _HEADS),
                  ((0, 0), (0, 16 - _HEADS)))
    e1 = jnp.pad(jnp.kron(eye8, jnp.ones((1, _HID), f32)),
                 ((0, 16 - _HEADS), (0, 0)))
    e2 = jnp.pad(jnp.ones((1, _OUT), f32), ((0, 15), (0, 0)))
    m2s = jnp.pad(a_src2.reshape(_OUT, 1).astype(f32), ((0, 0), (0, 15)))
    m2d = jnp.pad(a_dst2.reshape(_OUT, 1).astype(f32), ((0, 0), (0, 15)))

    grid = _N // _MROWS
    c1 = _HEADS * _HID

    tc_params = pltpu.CompilerParams(
        dimension_semantics=("parallel",))
    h1, as1, ad1 = pl.pallas_call(
        _tc1_body,
        grid=(grid,),
        compiler_params=tc_params,
        in_specs=[
            pl.BlockSpec((_MROWS, _IN), lambda i: (i, 0)),
            pl.BlockSpec((_IN, c1), lambda i: (0, 0)),
            pl.BlockSpec((c1, 16), lambda i: (0, 0)),
            pl.BlockSpec((c1, 16), lambda i: (0, 0)),
        ],
        out_specs=[
            pl.BlockSpec((_MROWS, c1), lambda i: (i, 0)),
            pl.BlockSpec((_MROWS, 16), lambda i: (i, 0)),
            pl.BlockSpec((_MROWS, 16), lambda i: (i, 0)),
        ],
        out_shape=[
            jax.ShapeDtypeStruct((_N, c1), f32),
            jax.ShapeDtypeStruct((_N, 16), f32),
            jax.ShapeDtypeStruct((_N, 16), f32),
        ],
    )(x, W1, m1s, m1d)

    acc1, den1 = _edge_kernel(c1, _HEADS, 200)(h1, as1, ad1, src, dst)

    h2, as2, ad2 = pl.pallas_call(
        _tc2_body,
        grid=(grid,),
        compiler_params=tc_params,
        in_specs=[
            pl.BlockSpec((_NC, _MROWS, c1), lambda i: (0, i, 0)),
            pl.BlockSpec((_NC, _MROWS, 16), lambda i: (0, i, 0)),
            pl.BlockSpec((16, c1), lambda i: (0, 0)),
            pl.BlockSpec((1, c1), lambda i: (0, 0)),
            pl.BlockSpec((c1, _OUT), lambda i: (0, 0)),
            pl.BlockSpec((_OUT, 16), lambda i: (0, 0)),
            pl.BlockSpec((_OUT, 16), lambda i: (0, 0)),
        ],
        out_specs=[
            pl.BlockSpec((_MROWS, _OUT), lambda i: (i, 0)),
            pl.BlockSpec((_MROWS, 16), lambda i: (i, 0)),
            pl.BlockSpec((_MROWS, 16), lambda i: (i, 0)),
        ],
        out_shape=[
            jax.ShapeDtypeStruct((_N, _OUT), f32),
            jax.ShapeDtypeStruct((_N, 16), f32),
            jax.ShapeDtypeStruct((_N, 16), f32),
        ],
    )(acc1, den1, e1, b1.reshape(1, c1), W2, m2s, m2d)

    acc2, den2 = _edge_kernel(_OUT, 1, 400)(h2, as2, ad2, src, dst)

    out = pl.pallas_call(
        _tc3_body,
        grid=(grid,),
        compiler_params=tc_params,
        in_specs=[
            pl.BlockSpec((_NC, _MROWS, _OUT), lambda i: (0, i, 0)),
            pl.BlockSpec((_NC, _MROWS, 16), lambda i: (0, i, 0)),
            pl.BlockSpec((16, _OUT), lambda i: (0, 0)),
            pl.BlockSpec((1, _OUT), lambda i: (0, 0)),
        ],
        out_specs=pl.BlockSpec((_MROWS, _OUT), lambda i: (i, 0)),
        out_shape=jax.ShapeDtypeStruct((_N, _OUT), f32),
    )(acc2, den2, e2, b2.reshape(1, _OUT))

    return out
